# baseline (speedup 1.0000x reference)
import functools

import jax
import jax.numpy as jnp
from jax import lax
from jax.experimental import pallas as pl
from jax.experimental.pallas import tpu as pltpu
from jax.experimental.pallas import tpu_sc as plsc

N = 10000
D = 256
H = 128
E = 160000
NC, NS = 2, 16
CHUNK = 128
CH_PER_TILE = 80
NBUF = 2
G = 8
NG = CH_PER_TILE // G
EPT = CH_PER_TILE * CHUNK
EP = NS * EPT
ROWS2 = EP // CHUNK
TRASH = N
ACCROWS = 10112
ROWS_PER_TILE = ACCROWS // NS
BLK = 1000


def _sc_segment_sum(xr, src2, dst2):
    mesh = plsc.VectorSubcoreMesh(
        core_axis_name="c", subcore_axis_name="s", num_cores=NC, num_subcores=NS
    )

    @functools.partial(
        pl.kernel,
        out_type=(
            jax.ShapeDtypeStruct((NC, ACCROWS, H), jnp.float32),
            jax.ShapeDtypeStruct((ACCROWS,), jnp.float32),
        ),
        mesh=mesh,
        scratch_types=[
            [pltpu.VMEM((G * CHUNK,), jnp.int32) for _ in range(2)],
            [pltpu.VMEM((G * CHUNK,), jnp.int32) for _ in range(2)],
            [pltpu.VMEM((CHUNK, H), jnp.float32) for _ in range(NBUF)],
            pltpu.VMEM((G * CHUNK,), jnp.float32),
            pltpu.VMEM_SHARED((ACCROWS, H), jnp.float32),
            pltpu.VMEM_SHARED((ACCROWS,), jnp.float32),
            [pltpu.SemaphoreType.DMA for _ in range(NBUF)],
            [pltpu.SemaphoreType.DMA for _ in range(NBUF)],
            pltpu.SemaphoreType.DMA,
            pltpu.SemaphoreType.DMA,
            pltpu.SemaphoreType.DMA,
        ],
        compiler_params=pltpu.CompilerParams(needs_layout_passes=False),
    )
    def k(xr_hbm, src_hbm, dst_hbm, zrow_hbm, zdeg_hbm, out_hbm, outd_hbm,
          src_v, dst_v, rows, ones_v, acc, deg, gsems, ssems, sisem, disem,
          dsem):
        c = lax.axis_index("c")
        s = lax.axis_index("s")
        pltpu.sync_copy(zrow_hbm,
                        acc.at[pl.ds(s * ROWS_PER_TILE, ROWS_PER_TILE)])

        @pl.when(s == 0)
        def _():
            pltpu.sync_copy(zdeg_hbm, deg)
        for i in range(G * CHUNK // 16):
            ones_v[pl.ds(i * 16, 16)] = jnp.ones((16,), jnp.float32)

        GW = G * CHUNK
        sbase = c * EP + s * EPT
        dbase = s * EPT

        def idx_group(g):
            buf = g % 2
            sc = pltpu.make_async_copy(
                src_hbm.at[pl.ds(sbase + g * GW, GW)], src_v[buf], sisem)
            dc = pltpu.make_async_copy(
                dst_hbm.at[pl.ds(dbase + g * GW, GW)], dst_v[buf], disem)
            return sc, dc

        def load_idx(g):
            sc, dc = idx_group(g)
            sc.start()
            dc.start()

        def wait_idx(g):
            sc, dc = idx_group(g)
            sc.wait()
            dc.wait()

        def src_row(j):
            return src_v[(j // G) % 2].at[pl.ds((j % G) * CHUNK, CHUNK)]

        def dst_row(j):
            return dst_v[(j // G) % 2].at[pl.ds((j % G) * CHUNK, CHUNK)]

        def gather(j):
            b = j % NBUF
            return pltpu.make_async_copy(xr_hbm.at[src_row(j)], rows[b],
                                         gsems[b])

        def scatter_start(j):
            b = j % NBUF
            pltpu.async_copy(rows[b], acc.at[dst_row(j)], ssems[b], add=True)

        def scatter_wait(j):
            b = j % NBUF
            pltpu.make_async_copy(rows[b], acc.at[dst_row(j)],
                                  ssems[b]).wait()

        def deg_start(g):
            pltpu.async_copy(ones_v, deg.at[dst_v[g % 2]], dsem, add=True)

        def deg_wait(g):
            pltpu.make_async_copy(ones_v, deg.at[dst_v[g % 2]], dsem).wait()

        wait0 = idx_group(0)
        load_idx(0)
        wait0[0].wait()
        wait0[1].wait()
        load_idx(1)
        gather(0).start()
        gather(1).start()
        plsc.subcore_barrier()

        @pl.when(c == 0)
        def _():
            deg_start(0)

        for j in range(CH_PER_TILE):
            g, r = divmod(j, G)
            if r == 0 and 1 <= g < NG - 1:
                @pl.when(c == 0)
                def _():
                    deg_wait(g - 1)

                load_idx(g + 1)
            if r == G - 2 and g + 1 < NG:
                wait_idx(g + 1)

                @pl.when(c == 0)
                def _():
                    deg_start(g + 1)

            gather(j).wait()
            scatter_start(j)
            scatter_wait(j)
            if j + NBUF < CH_PER_TILE:
                gather(j + NBUF).start()

        @pl.when(c == 0)
        def _():
            deg_wait(NG - 2)
            deg_wait(NG - 1)

        plsc.subcore_barrier()
        pltpu.sync_copy(acc.at[pl.ds(s * ROWS_PER_TILE, ROWS_PER_TILE)],
                        out_hbm.at[c, pl.ds(s * ROWS_PER_TILE, ROWS_PER_TILE)])

        @pl.when(jnp.logical_and(c == 0, s == 0))
        def _():
            pltpu.sync_copy(deg, outd_hbm)

    zrow = jnp.zeros((ROWS_PER_TILE, H), jnp.float32)
    zdeg = jnp.zeros((ACCROWS,), jnp.float32)
    return k(xr, src2, dst2, zrow, zdeg)


def _tc_finish_body(x_ref, sa_ref, sb_ref, deg_ref, wm_ref, ws_ref, b_ref,
                    g_ref, be_ref, o_ref):
    inv = 1.0 / jnp.maximum(deg_ref[...], 1.0)
    ssum = jnp.concatenate([sa_ref[0], sb_ref[0]], axis=1)
    h = (
        lax.dot_general(x_ref[...], ws_ref[...], (((1,), (1,)), ((), ())),
                        preferred_element_type=jnp.float32,
                        precision=lax.Precision.HIGHEST)
        + lax.dot_general(ssum, wm_ref[...], (((1,), (1,)), ((), ())),
                          preferred_element_type=jnp.float32,
                          precision=lax.Precision.HIGHEST) * inv
        + b_ref[...]
    )
    h = jnp.maximum(h, 0.0)
    mu = jnp.mean(h, axis=1, keepdims=True)
    d = h - mu
    var = jnp.mean(d * d, axis=1, keepdims=True)
    o_ref[...] = d * lax.rsqrt(var + 1e-5) * g_ref[...] + be_ref[...]


def _tc_finish(x, s3, deg, W_msg, W_self, b_self, gamma, beta):
    return pl.pallas_call(
        _tc_finish_body,
        grid=(N // BLK,),
        in_specs=[
            pl.BlockSpec((BLK, D), lambda j: (j, 0)),
            pl.BlockSpec((1, BLK, H), lambda j: (0, j, 0)),
            pl.BlockSpec((1, BLK, H), lambda j: (1, j, 0)),
            pl.BlockSpec((BLK, 1), lambda j: (j, 0)),
            pl.BlockSpec((D, D), lambda j: (0, 0)),
            pl.BlockSpec((D, D), lambda j: (0, 0)),
            pl.BlockSpec((1, D), lambda j: (0, 0)),
            pl.BlockSpec((1, D), lambda j: (0, 0)),
            pl.BlockSpec((1, D), lambda j: (0, 0)),
        ],
        out_specs=pl.BlockSpec((BLK, D), lambda j: (j, 0)),
        out_shape=jax.ShapeDtypeStruct((N, D), jnp.float32),
    )(x, s3, s3, deg, W_msg, W_self, b_self.reshape(1, D),
      gamma.reshape(1, D), beta.reshape(1, D))


def kernel(x, edge_index, W_msg, W_self, b_self, gamma, beta):
    src = edge_index[0].astype(jnp.int32)
    dst = edge_index[1].astype(jnp.int32)
    pad = EP - E
    src_p = jnp.concatenate([src, jnp.zeros((pad,), jnp.int32)])
    dst_p = jnp.concatenate([dst, jnp.full((pad,), TRASH, jnp.int32)])
    src2 = jnp.concatenate([2 * src_p, 2 * src_p + 1])
    dst2 = dst_p

    xr = x.reshape(2 * N, H)
    s3, degfull = _sc_segment_sum(xr, src2, dst2)
    deg = degfull[:N].reshape(N, 1)
    return _tc_finish(x, s3, deg, W_msg, W_self, b_self, gamma, beta)


# speedup vs baseline: 1.0314x; 1.0314x over previous
import functools

import jax
import jax.numpy as jnp
from jax import lax
from jax.experimental import pallas as pl
from jax.experimental.pallas import tpu as pltpu
from jax.experimental.pallas import tpu_sc as plsc

N = 10000
D = 256
H = 128
E = 160000
NC, NS = 2, 16
CHUNK = 128
CH_PER_TILE = 80
NBUF = 2
G = 8
NG = CH_PER_TILE // G
EPT = CH_PER_TILE * CHUNK
EP = NS * EPT
ROWS2 = EP // CHUNK
TRASH = N
ACCROWS = 10112
ROWS_PER_TILE = ACCROWS // NS
BLK = 1000


def _sc_segment_sum(xr, src2, dst2):
    mesh = plsc.VectorSubcoreMesh(
        core_axis_name="c", subcore_axis_name="s", num_cores=NC, num_subcores=NS
    )

    @functools.partial(
        pl.kernel,
        out_type=(
            jax.ShapeDtypeStruct((NC, ACCROWS, H), jnp.float32),
            jax.ShapeDtypeStruct((ACCROWS,), jnp.float32),
        ),
        mesh=mesh,
        scratch_types=[
            [pltpu.VMEM((G * CHUNK,), jnp.int32) for _ in range(2)],
            [pltpu.VMEM((G * CHUNK,), jnp.int32) for _ in range(2)],
            [pltpu.VMEM((CHUNK, H), jnp.float32) for _ in range(NBUF)],
            pltpu.VMEM((G * CHUNK,), jnp.float32),
            pltpu.VMEM_SHARED((ACCROWS, H), jnp.float32),
            pltpu.VMEM_SHARED((ACCROWS,), jnp.float32),
            [pltpu.SemaphoreType.DMA for _ in range(NBUF)],
            [pltpu.SemaphoreType.DMA for _ in range(NBUF)],
            pltpu.SemaphoreType.DMA,
            pltpu.SemaphoreType.DMA,
            pltpu.SemaphoreType.DMA,
            pltpu.SemaphoreType.DMA,
        ],
        compiler_params=pltpu.CompilerParams(needs_layout_passes=False),
    )
    def k(xr_hbm, src_hbm, dst_hbm, zrow_hbm, zdeg_hbm, out_hbm, outd_hbm,
          src_v, dst_v, rows, ones_v, acc, deg, gsems, ssems, sisem, disem,
          dsem, zsem):
        c = lax.axis_index("c")
        s = lax.axis_index("s")
        zcopy = pltpu.make_async_copy(
            zrow_hbm, acc.at[pl.ds(s * ROWS_PER_TILE, ROWS_PER_TILE)], zsem)
        zcopy.start()

        @pl.when(s == 0)
        def _():
            pltpu.sync_copy(zdeg_hbm, deg)
        for i in range(G * CHUNK // 16):
            ones_v[pl.ds(i * 16, 16)] = jnp.ones((16,), jnp.float32)

        GW = G * CHUNK
        sbase = c * EP + s * EPT
        dbase = s * EPT

        def idx_group(g):
            buf = g % 2
            sc = pltpu.make_async_copy(
                src_hbm.at[pl.ds(sbase + g * GW, GW)], src_v[buf], sisem)
            dc = pltpu.make_async_copy(
                dst_hbm.at[pl.ds(dbase + g * GW, GW)], dst_v[buf], disem)
            return sc, dc

        def load_idx(g):
            sc, dc = idx_group(g)
            sc.start()
            dc.start()

        def wait_idx(g):
            sc, dc = idx_group(g)
            sc.wait()
            dc.wait()

        def src_row(j):
            return src_v[(j // G) % 2].at[pl.ds((j % G) * CHUNK, CHUNK)]

        def dst_row(j):
            return dst_v[(j // G) % 2].at[pl.ds((j % G) * CHUNK, CHUNK)]

        def gather(j):
            b = j % NBUF
            return pltpu.make_async_copy(xr_hbm.at[src_row(j)], rows[b],
                                         gsems[b])

        def scatter_start(j):
            b = j % NBUF
            pltpu.async_copy(rows[b], acc.at[dst_row(j)], ssems[b], add=True)

        def scatter_wait(j):
            b = j % NBUF
            pltpu.make_async_copy(rows[b], acc.at[dst_row(j)],
                                  ssems[b]).wait()

        def deg_start(g):
            pltpu.async_copy(ones_v, deg.at[dst_v[g % 2]], dsem, add=True)

        def deg_wait(g):
            pltpu.make_async_copy(ones_v, deg.at[dst_v[g % 2]], dsem).wait()

        wait0 = idx_group(0)
        load_idx(0)
        wait0[0].wait()
        wait0[1].wait()
        load_idx(1)
        gather(0).start()
        gather(1).start()
        zcopy.wait()
        plsc.subcore_barrier()

        @pl.when(c == 0)
        def _():
            deg_start(0)

        for j in range(CH_PER_TILE):
            g, r = divmod(j, G)
            if r == 0 and 1 <= g < NG - 1:
                @pl.when(c == 0)
                def _():
                    deg_wait(g - 1)

                load_idx(g + 1)
            if r == G - 2 and g + 1 < NG:
                wait_idx(g + 1)

                @pl.when(c == 0)
                def _():
                    deg_start(g + 1)

            gather(j).wait()
            scatter_start(j)
            scatter_wait(j)
            if j + NBUF < CH_PER_TILE:
                gather(j + NBUF).start()

        @pl.when(c == 0)
        def _():
            deg_wait(NG - 2)
            deg_wait(NG - 1)

        plsc.subcore_barrier()
        pltpu.sync_copy(acc.at[pl.ds(s * ROWS_PER_TILE, ROWS_PER_TILE)],
                        out_hbm.at[c, pl.ds(s * ROWS_PER_TILE, ROWS_PER_TILE)])

        @pl.when(jnp.logical_and(c == 0, s == 0))
        def _():
            pltpu.sync_copy(deg, outd_hbm)

    zrow = jnp.zeros((ROWS_PER_TILE, H), jnp.float32)
    zdeg = jnp.zeros((ACCROWS,), jnp.float32)
    return k(xr, src2, dst2, zrow, zdeg)


def _tc_self_body(x_ref, ws_ref, b_ref, o_ref):
    o_ref[...] = lax.dot_general(
        x_ref[...], ws_ref[...], (((1,), (1,)), ((), ())),
        preferred_element_type=jnp.float32,
        precision=lax.Precision.HIGHEST) + b_ref[...]


def _tc_self(x, W_self, b_self):
    return pl.pallas_call(
        _tc_self_body,
        grid=(N // BLK,),
        in_specs=[
            pl.BlockSpec((BLK, D), lambda j: (j, 0)),
            pl.BlockSpec((D, D), lambda j: (0, 0)),
            pl.BlockSpec((1, D), lambda j: (0, 0)),
        ],
        out_specs=pl.BlockSpec((BLK, D), lambda j: (j, 0)),
        out_shape=jax.ShapeDtypeStruct((N, D), jnp.float32),
    )(x, W_self, b_self.reshape(1, D))


def _tc_finish_body(xw_ref, sa_ref, sb_ref, deg_ref, wm_ref, g_ref, be_ref,
                    o_ref):
    inv = 1.0 / jnp.maximum(deg_ref[...], 1.0)
    ssum = jnp.concatenate([sa_ref[0], sb_ref[0]], axis=1)
    h = (
        xw_ref[...]
        + lax.dot_general(ssum, wm_ref[...], (((1,), (1,)), ((), ())),
                          preferred_element_type=jnp.float32,
                          precision=lax.Precision.HIGHEST) * inv
    )
    h = jnp.maximum(h, 0.0)
    mu = jnp.mean(h, axis=1, keepdims=True)
    d = h - mu
    var = jnp.mean(d * d, axis=1, keepdims=True)
    o_ref[...] = d * lax.rsqrt(var + 1e-5) * g_ref[...] + be_ref[...]


def _tc_finish(xw, s3, deg, W_msg, gamma, beta):
    return pl.pallas_call(
        _tc_finish_body,
        grid=(N // BLK,),
        in_specs=[
            pl.BlockSpec((BLK, D), lambda j: (j, 0)),
            pl.BlockSpec((1, BLK, H), lambda j: (0, j, 0)),
            pl.BlockSpec((1, BLK, H), lambda j: (1, j, 0)),
            pl.BlockSpec((BLK, 1), lambda j: (j, 0)),
            pl.BlockSpec((D, D), lambda j: (0, 0)),
            pl.BlockSpec((1, D), lambda j: (0, 0)),
            pl.BlockSpec((1, D), lambda j: (0, 0)),
        ],
        out_specs=pl.BlockSpec((BLK, D), lambda j: (j, 0)),
        out_shape=jax.ShapeDtypeStruct((N, D), jnp.float32),
    )(xw, s3, s3, deg, W_msg, gamma.reshape(1, D), beta.reshape(1, D))


def kernel(x, edge_index, W_msg, W_self, b_self, gamma, beta):
    src = edge_index[0].astype(jnp.int32)
    dst = edge_index[1].astype(jnp.int32)
    pad = EP - E
    src_p = jnp.concatenate([src, jnp.zeros((pad,), jnp.int32)])
    dst_p = jnp.concatenate([dst, jnp.full((pad,), TRASH, jnp.int32)])
    src2 = jnp.concatenate([2 * src_p, 2 * src_p + 1])
    dst2 = dst_p

    xr = x.reshape(2 * N, H)
    xw = _tc_self(x, W_self, b_self)
    s3, degfull = _sc_segment_sum(xr, src2, dst2)
    deg = degfull[:N].reshape(N, 1)
    return _tc_finish(xw, s3, deg, W_msg, gamma, beta)
